# Initial kernel scaffold; baseline (speedup 1.0000x reference)
import jax
import jax.numpy as jnp
from jax.experimental import pallas as pl


def kernel(x, V, U):
    raise NotImplementedError("write your pallas kernel here")


# speedup vs baseline: 7.2014x; 7.2014x over previous
import functools

import jax
import jax.numpy as jnp
from jax import lax
from jax.experimental import pallas as pl

D = 768
M = 64
B = 32
MB = M * B
K = 2
ALPHA = 1.0
EPS = 1e-08
TB = 256


def _block_kernel(x_ref, wv_ref, wu_ref, xout_ref, stats_ref, esum_ref, *, nblk, n_tok):
    i = pl.program_id(0)

    wv = wv_ref[...]
    wv = wv / jnp.maximum(jnp.sqrt(jnp.sum(wv * wv, axis=1, keepdims=True)), EPS)
    wu = wu_ref[...]
    wu = wu / jnp.maximum(jnp.sqrt(jnp.sum(wu * wu, axis=1, keepdims=True)), EPS)

    xb = x_ref[...]
    xn = xb / jnp.maximum(jnp.sqrt(jnp.sum(xb * xb, axis=1, keepdims=True)), EPS)

    wv16 = wv.astype(jnp.bfloat16)
    wu16 = wu.astype(jnp.bfloat16)
    h = lax.dot_general(xn.astype(jnp.bfloat16), wv16, (((1,), (1,)), ((), ())),
                        preferred_element_type=jnp.float32)

    sel = (lax.broadcasted_iota(jnp.int32, (MB, 128), 0) // B
           == lax.broadcasted_iota(jnp.int32, (MB, 128), 1)).astype(jnp.float32)
    energy = jnp.dot(h * h, sel, preferred_element_type=jnp.float32, precision=lax.Precision.HIGHEST)

    m_iota = lax.broadcasted_iota(jnp.int32, (TB, 128), 1)
    max1 = jnp.max(energy, axis=1, keepdims=True)
    idx1 = jnp.min(jnp.where(energy == max1, m_iota, MB), axis=1, keepdims=True)
    e2 = jnp.where(m_iota == idx1, -jnp.inf, energy)
    max2 = jnp.max(e2, axis=1, keepdims=True)
    idx2 = jnp.min(jnp.where(e2 == max2, m_iota, MB), axis=1, keepdims=True)

    c_expert = lax.broadcasted_iota(jnp.int32, (TB, MB), 1) // B
    maskf = ((c_expert == idx1) | (c_expert == idx2)).astype(jnp.float32)
    hm = h * maskf

    hm16 = hm.astype(jnp.bfloat16)
    x_hat = jnp.dot(hm16, wv16, preferred_element_type=jnp.float32)
    writes = jnp.dot(hm16, wu16, preferred_element_type=jnp.float32)
    recon = lax.dot_general(writes.astype(jnp.bfloat16), wu16, (((1,), (1,)), ((), ())),
                            preferred_element_type=jnp.float32)

    xo = xn + ALPHA * writes
    xo = xo / jnp.maximum(jnp.sqrt(jnp.sum(xo * xo, axis=1, keepdims=True)), EPS)
    xout_ref[...] = xo

    resid = xn - x_hat
    p_uncap = jnp.sum(resid * resid)
    p_cap = jnp.sum(max1 + max2)
    p_recon = jnp.sum(x_hat * x_hat)
    dr = recon - h
    p_w = jnp.sum(maskf * dr * dr)
    p_e = jnp.sum(energy, axis=0)[None, :]

    s_iota = lax.broadcasted_iota(jnp.int32, (1, 128), 1)
    partial = (jnp.where(s_iota == 0, p_uncap, 0.0)
               + jnp.where(s_iota == 1, p_cap, 0.0)
               + jnp.where(s_iota == 2, p_recon, 0.0)
               + jnp.where(s_iota == 3, p_w, 0.0))
    acc = jnp.where(i == 0, partial, stats_ref[...] + partial)
    es = jnp.where(i == 0, p_e, esum_ref[...] + p_e)
    stats_ref[...] = acc
    esum_ref[...] = es

    @pl.when(i == nblk - 1)
    def _finalize():
        n_f = jnp.float32(n_tok)
        uncap = acc[0, 0] / n_f
        cap = acc[0, 1] / n_f
        rec = acc[0, 2] / n_f
        w_loss = acc[0, 3] / jnp.float32(n_tok * K * B)
        avg_e = es / n_f
        denom = jnp.maximum(jnp.sum(avg_e), EPS)
        probs = jnp.maximum(avg_e / denom, EPS)
        plogp = jnp.where(s_iota < M, probs * jnp.log(probs), 0.0)
        ent = -jnp.sum(plogp) / jnp.log(jnp.float32(M))
        total = uncap + 0.5 * (1.0 - ent) + w_loss
        final = (jnp.where(s_iota == 0, total, 0.0)
                 + jnp.where(s_iota == 1, cap, 0.0)
                 + jnp.where(s_iota == 2, rec, 0.0)
                 + jnp.where(s_iota == 3, uncap, 0.0)
                 + jnp.where(s_iota == 4, ent, 0.0))
        stats_ref[...] = final


@jax.jit
def kernel(x, V, U):
    input_shape = x.shape
    n_tok = x.shape[0] * x.shape[1]
    x2d = x.reshape(n_tok, D)
    vr = jnp.transpose(V, (1, 2, 0)).reshape(MB, D)
    ur = jnp.transpose(U, (0, 2, 1)).reshape(MB, D)

    nblk = n_tok // TB
    body = functools.partial(_block_kernel, nblk=nblk, n_tok=n_tok)
    xout, stats, _ = pl.pallas_call(
        body,
        grid=(nblk,),
        in_specs=[
            pl.BlockSpec((TB, D), lambda i: (i, 0)),
            pl.BlockSpec((MB, D), lambda i: (0, 0)),
            pl.BlockSpec((MB, D), lambda i: (0, 0)),
        ],
        out_specs=[
            pl.BlockSpec((TB, D), lambda i: (i, 0)),
            pl.BlockSpec((1, 128), lambda i: (0, 0)),
            pl.BlockSpec((1, 128), lambda i: (0, 0)),
        ],
        out_shape=[
            jax.ShapeDtypeStruct((n_tok, D), jnp.float32),
            jax.ShapeDtypeStruct((1, 128), jnp.float32),
            jax.ShapeDtypeStruct((1, 128), jnp.float32),
        ],
    )(x2d, vr, ur)

    x_out = xout.reshape(input_shape)
    return (x_out, stats[0, 0], stats[0, 1], stats[0, 2], stats[0, 3], stats[0, 4])


# speedup vs baseline: 10.4669x; 1.4535x over previous
import functools

import jax
import jax.numpy as jnp
from jax import lax
from jax.experimental import pallas as pl
from jax.experimental.pallas import tpu as pltpu

D = 768
M = 64
B = 32
MB = M * B
K = 2
ALPHA = 1.0
EPS = 1e-08
TB = 512


def _block_kernel(x_ref, wv_ref, wu_ref, xout_ref, stats_ref, esum_ref,
                  wvu_s, wvt_s, wut_s, *, nblk, n_tok):
    i = pl.program_id(0)

    @pl.when(i == 0)
    def _prep():
        wv = wv_ref[...]
        wv = wv / jnp.maximum(jnp.sqrt(jnp.sum(wv * wv, axis=1, keepdims=True)), EPS)
        wu = wu_ref[...]
        wu = wu / jnp.maximum(jnp.sqrt(jnp.sum(wu * wu, axis=1, keepdims=True)), EPS)
        wv16 = wv.astype(jnp.bfloat16)
        wu16 = wu.astype(jnp.bfloat16)
        wvu_s[:, :D] = wv16
        wvu_s[:, D:] = wu16
        wvt_s[...] = wv16.T
        wut_s[...] = wu16.T

    xb = x_ref[...]
    xn = xb / jnp.maximum(jnp.sqrt(jnp.sum(xb * xb, axis=1, keepdims=True)), EPS)

    h = jnp.dot(xn.astype(jnp.bfloat16), wvt_s[...],
                preferred_element_type=jnp.float32)

    sel = (lax.broadcasted_iota(jnp.int32, (MB, 128), 0) // B
           == lax.broadcasted_iota(jnp.int32, (MB, 128), 1)).astype(jnp.bfloat16)
    hh = h * h
    hh_hi = hh.astype(jnp.bfloat16)
    hh_lo = (hh - hh_hi.astype(jnp.float32)).astype(jnp.bfloat16)
    energy = (jnp.dot(hh_hi, sel, preferred_element_type=jnp.float32)
              + jnp.dot(hh_lo, sel, preferred_element_type=jnp.float32))

    m_iota = lax.broadcasted_iota(jnp.int32, (TB, 128), 1)
    max1 = jnp.max(energy, axis=1, keepdims=True)
    idx1 = jnp.min(jnp.where(energy == max1, m_iota, MB), axis=1, keepdims=True)
    e2 = jnp.where(m_iota == idx1, -jnp.inf, energy)
    max2 = jnp.max(e2, axis=1, keepdims=True)
    idx2 = jnp.min(jnp.where(e2 == max2, m_iota, MB), axis=1, keepdims=True)

    c_expert = lax.broadcasted_iota(jnp.int32, (TB, MB), 1) // B
    maskf = ((c_expert == idx1) | (c_expert == idx2)).astype(jnp.float32)
    hm16 = (h * maskf).astype(jnp.bfloat16)

    xw = jnp.dot(hm16, wvu_s[...], preferred_element_type=jnp.float32)
    x_hat = xw[:, :D]
    writes = xw[:, D:]
    recon = jnp.dot(writes.astype(jnp.bfloat16), wut_s[...],
                    preferred_element_type=jnp.float32)

    xo = xn + ALPHA * writes
    xo = xo / jnp.maximum(jnp.sqrt(jnp.sum(xo * xo, axis=1, keepdims=True)), EPS)
    xout_ref[...] = xo

    resid = xn - x_hat
    p_uncap = jnp.sum(resid * resid)
    p_cap = jnp.sum(max1 + max2)
    p_recon = jnp.sum(x_hat * x_hat)
    dr = recon - h
    p_w = jnp.sum(maskf * dr * dr)
    p_e = jnp.sum(energy, axis=0)[None, :]

    s_iota = lax.broadcasted_iota(jnp.int32, (1, 128), 1)
    partial = (jnp.where(s_iota == 0, p_uncap, 0.0)
               + jnp.where(s_iota == 1, p_cap, 0.0)
               + jnp.where(s_iota == 2, p_recon, 0.0)
               + jnp.where(s_iota == 3, p_w, 0.0))
    acc = jnp.where(i == 0, partial, stats_ref[...] + partial)
    es = jnp.where(i == 0, p_e, esum_ref[...] + p_e)
    stats_ref[...] = acc
    esum_ref[...] = es

    @pl.when(i == nblk - 1)
    def _finalize():
        n_f = jnp.float32(n_tok)
        uncap = acc[0, 0] / n_f
        cap = acc[0, 1] / n_f
        rec = acc[0, 2] / n_f
        w_loss = acc[0, 3] / jnp.float32(n_tok * K * B)
        avg_e = es / n_f
        denom = jnp.maximum(jnp.sum(avg_e), EPS)
        probs = jnp.maximum(avg_e / denom, EPS)
        plogp = jnp.where(s_iota < M, probs * jnp.log(probs), 0.0)
        ent = -jnp.sum(plogp) / jnp.log(jnp.float32(M))
        total = uncap + 0.5 * (1.0 - ent) + w_loss
        final = (jnp.where(s_iota == 0, total, 0.0)
                 + jnp.where(s_iota == 1, cap, 0.0)
                 + jnp.where(s_iota == 2, rec, 0.0)
                 + jnp.where(s_iota == 3, uncap, 0.0)
                 + jnp.where(s_iota == 4, ent, 0.0))
        stats_ref[...] = final


@jax.jit
def kernel(x, V, U):
    input_shape = x.shape
    n_tok = x.shape[0] * x.shape[1]
    x2d = x.reshape(n_tok, D)
    vr = jnp.transpose(V, (1, 2, 0)).reshape(MB, D)
    ur = jnp.transpose(U, (0, 2, 1)).reshape(MB, D)

    nblk = n_tok // TB
    body = functools.partial(_block_kernel, nblk=nblk, n_tok=n_tok)
    xout, stats, _ = pl.pallas_call(
        body,
        grid=(nblk,),
        in_specs=[
            pl.BlockSpec((TB, D), lambda i: (i, 0)),
            pl.BlockSpec((MB, D), lambda i: (0, 0)),
            pl.BlockSpec((MB, D), lambda i: (0, 0)),
        ],
        out_specs=[
            pl.BlockSpec((TB, D), lambda i: (i, 0)),
            pl.BlockSpec((1, 128), lambda i: (0, 0)),
            pl.BlockSpec((1, 128), lambda i: (0, 0)),
        ],
        out_shape=[
            jax.ShapeDtypeStruct((n_tok, D), jnp.float32),
            jax.ShapeDtypeStruct((1, 128), jnp.float32),
            jax.ShapeDtypeStruct((1, 128), jnp.float32),
        ],
        scratch_shapes=[
            pltpu.VMEM((MB, 2 * D), jnp.bfloat16),
            pltpu.VMEM((D, MB), jnp.bfloat16),
            pltpu.VMEM((D, MB), jnp.bfloat16),
        ],
    )(x2d, vr, ur)

    x_out = xout.reshape(input_shape)
    return (x_out, stats[0, 0], stats[0, 1], stats[0, 2], stats[0, 3], stats[0, 4])


# speedup vs baseline: 10.8990x; 1.0413x over previous
import functools

import jax
import jax.numpy as jnp
from jax import lax
from jax.experimental import pallas as pl
from jax.experimental.pallas import tpu as pltpu

D = 768
M = 64
B = 32
MB = M * B
K = 2
ALPHA = 1.0
EPS = 1e-08
TB = 512


def _block_kernel(x_ref, wv_ref, wu_ref, xout_ref, stats_ref, esum_ref,
                  wvu_s, *, nblk, n_tok):
    i = pl.program_id(0)

    @pl.when(i == 0)
    def _prep():
        wv = wv_ref[...]
        wv = wv / jnp.maximum(jnp.sqrt(jnp.sum(wv * wv, axis=1, keepdims=True)), EPS)
        wu = wu_ref[...]
        wu = wu / jnp.maximum(jnp.sqrt(jnp.sum(wu * wu, axis=1, keepdims=True)), EPS)
        wvu_s[:, :D] = wv.astype(jnp.bfloat16)
        wvu_s[:, D:] = wu.astype(jnp.bfloat16)

    xb = x_ref[...]
    xn = xb / jnp.maximum(jnp.sqrt(jnp.sum(xb * xb, axis=1, keepdims=True)), EPS)

    h = lax.dot_general(xn.astype(jnp.bfloat16), wvu_s[:, :D],
                        (((1,), (1,)), ((), ())),
                        preferred_element_type=jnp.float32)

    sel = (lax.broadcasted_iota(jnp.int32, (MB, 128), 0) // B
           == lax.broadcasted_iota(jnp.int32, (MB, 128), 1)).astype(jnp.bfloat16)
    hh = h * h
    hh_hi = hh.astype(jnp.bfloat16)
    hh_lo = (hh - hh_hi.astype(jnp.float32)).astype(jnp.bfloat16)
    energy = (jnp.dot(hh_hi, sel, preferred_element_type=jnp.float32)
              + jnp.dot(hh_lo, sel, preferred_element_type=jnp.float32))

    m_iota = lax.broadcasted_iota(jnp.int32, (TB, 128), 1)
    max1 = jnp.max(energy, axis=1, keepdims=True)
    idx1 = jnp.min(jnp.where(energy == max1, m_iota, MB), axis=1, keepdims=True)
    e2 = jnp.where(m_iota == idx1, -jnp.inf, energy)
    max2 = jnp.max(e2, axis=1, keepdims=True)
    idx2 = jnp.min(jnp.where(e2 == max2, m_iota, MB), axis=1, keepdims=True)

    onehot2 = ((m_iota == idx1) | (m_iota == idx2)).astype(jnp.bfloat16)
    expand = (lax.broadcasted_iota(jnp.int32, (128, MB), 0)
              == lax.broadcasted_iota(jnp.int32, (128, MB), 1) // B
              ).astype(jnp.bfloat16)
    maskf = jnp.dot(onehot2, expand, preferred_element_type=jnp.float32)
    hm16 = (h * maskf).astype(jnp.bfloat16)

    xw = jnp.dot(hm16, wvu_s[...], preferred_element_type=jnp.float32)
    x_hat = xw[:, :D]
    writes = xw[:, D:]
    recon = lax.dot_general(writes.astype(jnp.bfloat16), wvu_s[:, D:],
                            (((1,), (1,)), ((), ())),
                            preferred_element_type=jnp.float32)

    xo = xn + ALPHA * writes
    xo = xo / jnp.maximum(jnp.sqrt(jnp.sum(xo * xo, axis=1, keepdims=True)), EPS)
    xout_ref[...] = xo

    resid = xn - x_hat
    p_uncap = jnp.sum(resid * resid)
    p_cap = jnp.sum(max1 + max2)
    p_recon = jnp.sum(x_hat * x_hat)
    dr = recon - h
    p_w = jnp.sum(maskf * dr * dr)
    p_e = jnp.sum(energy, axis=0)[None, :]

    s_iota = lax.broadcasted_iota(jnp.int32, (1, 128), 1)
    partial = (jnp.where(s_iota == 0, p_uncap, 0.0)
               + jnp.where(s_iota == 1, p_cap, 0.0)
               + jnp.where(s_iota == 2, p_recon, 0.0)
               + jnp.where(s_iota == 3, p_w, 0.0))
    acc = jnp.where(i == 0, partial, stats_ref[...] + partial)
    es = jnp.where(i == 0, p_e, esum_ref[...] + p_e)
    stats_ref[...] = acc
    esum_ref[...] = es

    @pl.when(i == nblk - 1)
    def _finalize():
        n_f = jnp.float32(n_tok)
        uncap = acc[0, 0] / n_f
        cap = acc[0, 1] / n_f
        rec = acc[0, 2] / n_f
        w_loss = acc[0, 3] / jnp.float32(n_tok * K * B)
        avg_e = es / n_f
        denom = jnp.maximum(jnp.sum(avg_e), EPS)
        probs = jnp.maximum(avg_e / denom, EPS)
        plogp = jnp.where(s_iota < M, probs * jnp.log(probs), 0.0)
        ent = -jnp.sum(plogp) / jnp.log(jnp.float32(M))
        total = uncap + 0.5 * (1.0 - ent) + w_loss
        final = (jnp.where(s_iota == 0, total, 0.0)
                 + jnp.where(s_iota == 1, cap, 0.0)
                 + jnp.where(s_iota == 2, rec, 0.0)
                 + jnp.where(s_iota == 3, uncap, 0.0)
                 + jnp.where(s_iota == 4, ent, 0.0))
        stats_ref[...] = final


@jax.jit
def kernel(x, V, U):
    input_shape = x.shape
    n_tok = x.shape[0] * x.shape[1]
    x2d = x.reshape(n_tok, D)
    vr = jnp.transpose(V, (1, 2, 0)).reshape(MB, D)
    ur = jnp.transpose(U, (0, 2, 1)).reshape(MB, D)

    nblk = n_tok // TB
    body = functools.partial(_block_kernel, nblk=nblk, n_tok=n_tok)
    xout, stats, _ = pl.pallas_call(
        body,
        grid=(nblk,),
        in_specs=[
            pl.BlockSpec((TB, D), lambda i: (i, 0)),
            pl.BlockSpec((MB, D), lambda i: (0, 0)),
            pl.BlockSpec((MB, D), lambda i: (0, 0)),
        ],
        out_specs=[
            pl.BlockSpec((TB, D), lambda i: (i, 0)),
            pl.BlockSpec((1, 128), lambda i: (0, 0)),
            pl.BlockSpec((1, 128), lambda i: (0, 0)),
        ],
        out_shape=[
            jax.ShapeDtypeStruct((n_tok, D), jnp.float32),
            jax.ShapeDtypeStruct((1, 128), jnp.float32),
            jax.ShapeDtypeStruct((1, 128), jnp.float32),
        ],
        scratch_shapes=[
            pltpu.VMEM((MB, 2 * D), jnp.bfloat16),
        ],
    )(x2d, vr, ur)

    x_out = xout.reshape(input_shape)
    return (x_out, stats[0, 0], stats[0, 1], stats[0, 2], stats[0, 3], stats[0, 4])


# speedup vs baseline: 11.1435x; 1.0224x over previous
import functools

import jax
import jax.numpy as jnp
from jax import lax
from jax.experimental import pallas as pl
from jax.experimental.pallas import tpu as pltpu

D = 768
M = 64
B = 32
MB = M * B
K = 2
ALPHA = 1.0
EPS = 1e-08
TB = 1024
HB = 512


def _half(xb, wvu_s):
    xn = xb / jnp.maximum(jnp.sqrt(jnp.sum(xb * xb, axis=1, keepdims=True)), EPS)

    h = lax.dot_general(xn.astype(jnp.bfloat16), wvu_s[:, :D],
                        (((1,), (1,)), ((), ())),
                        preferred_element_type=jnp.float32)

    sel = (lax.broadcasted_iota(jnp.int32, (MB, 128), 0) // B
           == lax.broadcasted_iota(jnp.int32, (MB, 128), 1)).astype(jnp.bfloat16)
    hh = h * h
    hh_hi = hh.astype(jnp.bfloat16)
    hh_lo = (hh - hh_hi.astype(jnp.float32)).astype(jnp.bfloat16)
    energy = (jnp.dot(hh_hi, sel, preferred_element_type=jnp.float32)
              + jnp.dot(hh_lo, sel, preferred_element_type=jnp.float32))

    m_iota = lax.broadcasted_iota(jnp.int32, (HB, 128), 1)
    max1 = jnp.max(energy, axis=1, keepdims=True)
    idx1 = jnp.min(jnp.where(energy == max1, m_iota, MB), axis=1, keepdims=True)
    e2 = jnp.where(m_iota == idx1, -jnp.inf, energy)
    max2 = jnp.max(e2, axis=1, keepdims=True)
    idx2 = jnp.min(jnp.where(e2 == max2, m_iota, MB), axis=1, keepdims=True)

    onehot2 = ((m_iota == idx1) | (m_iota == idx2)).astype(jnp.bfloat16)
    expand = (lax.broadcasted_iota(jnp.int32, (128, MB), 0)
              == lax.broadcasted_iota(jnp.int32, (128, MB), 1) // B
              ).astype(jnp.bfloat16)
    maskf = jnp.dot(onehot2, expand, preferred_element_type=jnp.float32)
    hm16 = (h * maskf).astype(jnp.bfloat16)

    xw = jnp.dot(hm16, wvu_s[...], preferred_element_type=jnp.float32)
    x_hat = xw[:, :D]
    writes = xw[:, D:]
    recon = lax.dot_general(writes.astype(jnp.bfloat16), wvu_s[:, D:],
                            (((1,), (1,)), ((), ())),
                            preferred_element_type=jnp.float32)

    xo = xn + ALPHA * writes
    xo = xo / jnp.maximum(jnp.sqrt(jnp.sum(xo * xo, axis=1, keepdims=True)), EPS)

    resid = xn - x_hat
    p_uncap = jnp.sum(resid * resid)
    p_cap = jnp.sum(max1 + max2)
    p_recon = jnp.sum(x_hat * x_hat)
    dr = recon - h
    p_w = jnp.sum(maskf * dr * dr)
    p_e = jnp.sum(energy, axis=0)[None, :]
    return xo, p_uncap, p_cap, p_recon, p_w, p_e


def _block_kernel(x_ref, wv_ref, wu_ref, xout_ref, stats_ref, esum_ref,
                  wvu_s, *, nblk, n_tok):
    i = pl.program_id(0)

    @pl.when(i == 0)
    def _prep():
        wv = wv_ref[...]
        wv = wv / jnp.maximum(jnp.sqrt(jnp.sum(wv * wv, axis=1, keepdims=True)), EPS)
        wu = wu_ref[...]
        wu = wu / jnp.maximum(jnp.sqrt(jnp.sum(wu * wu, axis=1, keepdims=True)), EPS)
        wvu_s[:, :D] = wv.astype(jnp.bfloat16)
        wvu_s[:, D:] = wu.astype(jnp.bfloat16)

    xo0, u0, c0, r0, w0, e0 = _half(x_ref[:HB, :], wvu_s)
    xo1, u1, c1, r1, w1, e1 = _half(x_ref[HB:, :], wvu_s)
    xout_ref[:HB, :] = xo0
    xout_ref[HB:, :] = xo1
    p_uncap = u0 + u1
    p_cap = c0 + c1
    p_recon = r0 + r1
    p_w = w0 + w1
    p_e = e0 + e1

    s_iota = lax.broadcasted_iota(jnp.int32, (1, 128), 1)
    partial = (jnp.where(s_iota == 0, p_uncap, 0.0)
               + jnp.where(s_iota == 1, p_cap, 0.0)
               + jnp.where(s_iota == 2, p_recon, 0.0)
               + jnp.where(s_iota == 3, p_w, 0.0))
    acc = jnp.where(i == 0, partial, stats_ref[...] + partial)
    es = jnp.where(i == 0, p_e, esum_ref[...] + p_e)
    stats_ref[...] = acc
    esum_ref[...] = es

    @pl.when(i == nblk - 1)
    def _finalize():
        n_f = jnp.float32(n_tok)
        uncap = acc[0, 0] / n_f
        cap = acc[0, 1] / n_f
        rec = acc[0, 2] / n_f
        w_loss = acc[0, 3] / jnp.float32(n_tok * K * B)
        avg_e = es / n_f
        denom = jnp.maximum(jnp.sum(avg_e), EPS)
        probs = jnp.maximum(avg_e / denom, EPS)
        plogp = jnp.where(s_iota < M, probs * jnp.log(probs), 0.0)
        ent = -jnp.sum(plogp) / jnp.log(jnp.float32(M))
        total = uncap + 0.5 * (1.0 - ent) + w_loss
        final = (jnp.where(s_iota == 0, total, 0.0)
                 + jnp.where(s_iota == 1, cap, 0.0)
                 + jnp.where(s_iota == 2, rec, 0.0)
                 + jnp.where(s_iota == 3, uncap, 0.0)
                 + jnp.where(s_iota == 4, ent, 0.0))
        stats_ref[...] = final


@jax.jit
def kernel(x, V, U):
    input_shape = x.shape
    n_tok = x.shape[0] * x.shape[1]
    x2d = x.reshape(n_tok, D)
    vr = jnp.transpose(V, (1, 2, 0)).reshape(MB, D)
    ur = jnp.transpose(U, (0, 2, 1)).reshape(MB, D)

    nblk = n_tok // TB
    body = functools.partial(_block_kernel, nblk=nblk, n_tok=n_tok)
    xout, stats, _ = pl.pallas_call(
        body,
        grid=(nblk,),
        in_specs=[
            pl.BlockSpec((TB, D), lambda i: (i, 0)),
            pl.BlockSpec((MB, D), lambda i: (0, 0)),
            pl.BlockSpec((MB, D), lambda i: (0, 0)),
        ],
        out_specs=[
            pl.BlockSpec((TB, D), lambda i: (i, 0)),
            pl.BlockSpec((1, 128), lambda i: (0, 0)),
            pl.BlockSpec((1, 128), lambda i: (0, 0)),
        ],
        out_shape=[
            jax.ShapeDtypeStruct((n_tok, D), jnp.float32),
            jax.ShapeDtypeStruct((1, 128), jnp.float32),
            jax.ShapeDtypeStruct((1, 128), jnp.float32),
        ],
        scratch_shapes=[
            pltpu.VMEM((MB, 2 * D), jnp.bfloat16),
        ],
    )(x2d, vr, ur)

    x_out = xout.reshape(input_shape)
    return (x_out, stats[0, 0], stats[0, 1], stats[0, 2], stats[0, 3], stats[0, 4])


# speedup vs baseline: 12.1509x; 1.0904x over previous
import functools

import jax
import jax.numpy as jnp
from jax import lax
from jax.experimental import pallas as pl
from jax.experimental.pallas import tpu as pltpu

D = 768
M = 64
B = 32
MB = M * B
K = 2
ALPHA = 1.0
EPS = 1e-08
TB = 1024
HB = 256


def _half(xb, wvu_s, sel_s):
    xn = xb / jnp.maximum(jnp.sqrt(jnp.sum(xb * xb, axis=1, keepdims=True)), EPS)

    h = lax.dot_general(xn.astype(jnp.bfloat16), wvu_s[:, :D],
                        (((1,), (1,)), ((), ())),
                        preferred_element_type=jnp.float32)

    sel = sel_s[...]
    hh = h * h
    hh_hi = hh.astype(jnp.bfloat16)
    hh_lo = (hh - hh_hi.astype(jnp.float32)).astype(jnp.bfloat16)
    energy = (jnp.dot(hh_hi, sel, preferred_element_type=jnp.float32)
              + jnp.dot(hh_lo, sel, preferred_element_type=jnp.float32))

    m_iota = lax.broadcasted_iota(jnp.int32, (HB, 128), 1)
    max1 = jnp.max(energy, axis=1, keepdims=True)
    idx1 = jnp.min(jnp.where(energy == max1, m_iota, MB), axis=1, keepdims=True)
    e2 = jnp.where(m_iota == idx1, -jnp.inf, energy)
    max2 = jnp.max(e2, axis=1, keepdims=True)
    idx2 = jnp.min(jnp.where(e2 == max2, m_iota, MB), axis=1, keepdims=True)

    onehot2 = ((m_iota == idx1) | (m_iota == idx2)).astype(jnp.bfloat16)
    maskf = lax.dot_general(onehot2, sel, (((1,), (1,)), ((), ())),
                            preferred_element_type=jnp.float32)
    hm16 = (h * maskf).astype(jnp.bfloat16)

    xw = jnp.dot(hm16, wvu_s[...], preferred_element_type=jnp.float32)
    x_hat = xw[:, :D]
    writes = xw[:, D:]
    recon = lax.dot_general(writes.astype(jnp.bfloat16), wvu_s[:, D:],
                            (((1,), (1,)), ((), ())),
                            preferred_element_type=jnp.float32)

    xo = xn + ALPHA * writes
    xo = xo / jnp.maximum(jnp.sqrt(jnp.sum(xo * xo, axis=1, keepdims=True)), EPS)

    resid = xn - x_hat
    p_uncap = jnp.sum(resid * resid)
    p_cap = jnp.sum(max1 + max2)
    p_recon = jnp.sum(x_hat * x_hat)
    dr = recon - h
    p_w = jnp.sum(maskf * dr * dr)
    p_e = jnp.sum(energy, axis=0)[None, :]
    return xo, p_uncap, p_cap, p_recon, p_w, p_e


def _block_kernel(x_ref, wv_ref, wu_ref, xout_ref, stats_ref, esum_ref,
                  wvu_s, sel_s, *, nblk, n_tok):
    i = pl.program_id(0)

    @pl.when(i == 0)
    def _prep():
        sel_s[...] = (lax.broadcasted_iota(jnp.int32, (MB, 128), 0) // B
                      == lax.broadcasted_iota(jnp.int32, (MB, 128), 1)
                      ).astype(jnp.bfloat16)
        wv = wv_ref[...]
        wv = wv / jnp.maximum(jnp.sqrt(jnp.sum(wv * wv, axis=1, keepdims=True)), EPS)
        wu = wu_ref[...]
        wu = wu / jnp.maximum(jnp.sqrt(jnp.sum(wu * wu, axis=1, keepdims=True)), EPS)
        wvu_s[:, :D] = wv.astype(jnp.bfloat16)
        wvu_s[:, D:] = wu.astype(jnp.bfloat16)

    parts = []
    for q in range(TB // HB):
        xo, *ps = _half(x_ref[q * HB:(q + 1) * HB, :], wvu_s, sel_s)
        xout_ref[q * HB:(q + 1) * HB, :] = xo
        parts.append(ps)
    p_uncap, p_cap, p_recon, p_w, p_e = [sum(t[j] for t in parts)
                                         for j in range(5)]

    s_iota = lax.broadcasted_iota(jnp.int32, (1, 128), 1)
    partial = (jnp.where(s_iota == 0, p_uncap, 0.0)
               + jnp.where(s_iota == 1, p_cap, 0.0)
               + jnp.where(s_iota == 2, p_recon, 0.0)
               + jnp.where(s_iota == 3, p_w, 0.0))
    acc = jnp.where(i == 0, partial, stats_ref[...] + partial)
    es = jnp.where(i == 0, p_e, esum_ref[...] + p_e)
    stats_ref[...] = acc
    esum_ref[...] = es

    @pl.when(i == nblk - 1)
    def _finalize():
        n_f = jnp.float32(n_tok)
        uncap = acc[0, 0] / n_f
        cap = acc[0, 1] / n_f
        rec = acc[0, 2] / n_f
        w_loss = acc[0, 3] / jnp.float32(n_tok * K * B)
        avg_e = es / n_f
        denom = jnp.maximum(jnp.sum(avg_e), EPS)
        probs = jnp.maximum(avg_e / denom, EPS)
        plogp = jnp.where(s_iota < M, probs * jnp.log(probs), 0.0)
        ent = -jnp.sum(plogp) / jnp.log(jnp.float32(M))
        total = uncap + 0.5 * (1.0 - ent) + w_loss
        final = (jnp.where(s_iota == 0, total, 0.0)
                 + jnp.where(s_iota == 1, cap, 0.0)
                 + jnp.where(s_iota == 2, rec, 0.0)
                 + jnp.where(s_iota == 3, uncap, 0.0)
                 + jnp.where(s_iota == 4, ent, 0.0))
        stats_ref[...] = final


@jax.jit
def kernel(x, V, U):
    input_shape = x.shape
    n_tok = x.shape[0] * x.shape[1]
    x2d = x.reshape(n_tok, D)
    vr = jnp.transpose(V, (1, 2, 0)).reshape(MB, D)
    ur = jnp.transpose(U, (0, 2, 1)).reshape(MB, D)

    nblk = n_tok // TB
    body = functools.partial(_block_kernel, nblk=nblk, n_tok=n_tok)
    xout, stats, _ = pl.pallas_call(
        body,
        grid=(nblk,),
        in_specs=[
            pl.BlockSpec((TB, D), lambda i: (i, 0)),
            pl.BlockSpec((MB, D), lambda i: (0, 0)),
            pl.BlockSpec((MB, D), lambda i: (0, 0)),
        ],
        out_specs=[
            pl.BlockSpec((TB, D), lambda i: (i, 0)),
            pl.BlockSpec((1, 128), lambda i: (0, 0)),
            pl.BlockSpec((1, 128), lambda i: (0, 0)),
        ],
        out_shape=[
            jax.ShapeDtypeStruct((n_tok, D), jnp.float32),
            jax.ShapeDtypeStruct((1, 128), jnp.float32),
            jax.ShapeDtypeStruct((1, 128), jnp.float32),
        ],
        scratch_shapes=[
            pltpu.VMEM((MB, 2 * D), jnp.bfloat16),
            pltpu.VMEM((MB, 128), jnp.bfloat16),
        ],
    )(x2d, vr, ur)

    x_out = xout.reshape(input_shape)
    return (x_out, stats[0, 0], stats[0, 1], stats[0, 2], stats[0, 3], stats[0, 4])
